# baseline (speedup 1.0000x reference)
import functools

import jax
import jax.numpy as jnp
from jax import lax
from jax.experimental import pallas as pl
from jax.experimental.pallas import tpu as pltpu
from jax.experimental.pallas import tpu_sc as plsc

B = 4096
L = 50
H = 64
N = B * L
CHUNK = 128
NCHUNK = N // CHUNK
NC = 2
NS = 16
NW = NC * NS
CPW = NCHUNK // NW
IPW = N // NW
NBUF = 4


@functools.partial(
    pl.kernel,
    out_type=jax.ShapeDtypeStruct((N, 2 * H), jnp.float32),
    mesh=plsc.VectorSubcoreMesh(core_axis_name="c", subcore_axis_name="s"),
    compiler_params=pltpu.CompilerParams(use_tc_tiling_on_sc=False),
    scratch_types=[
        pltpu.VMEM((IPW,), jnp.int32),
        pltpu.VMEM((IPW,), jnp.int32),
        pltpu.VMEM((NBUF, CHUNK, H), jnp.float32),
        pltpu.VMEM((NBUF, CHUNK, H), jnp.float32),
        pltpu.SemaphoreType.DMA((NBUF,)),
        pltpu.SemaphoreType.DMA((NBUF,)),
    ],
)
def _emb_gather(form_idx_hbm, vice_idx_hbm, wform_hbm, wvice_hbm, out_hbm,
                fidx_v, vidx_v, frows, vrows, gsem, wsem):
    wid = lax.axis_index("s") * NC + lax.axis_index("c")
    base = wid * CPW
    pltpu.sync_copy(form_idx_hbm.at[pl.ds(wid * IPW, IPW)], fidx_v)
    pltpu.sync_copy(vice_idx_hbm.at[pl.ds(wid * IPW, IPW)], vidx_v)

    def fire_gather(v, b):
        pltpu.async_copy(wform_hbm.at[fidx_v.at[pl.ds(v * CHUNK, CHUNK)]],
                         frows.at[b], gsem.at[b])
        pltpu.async_copy(wvice_hbm.at[vidx_v.at[pl.ds(v * CHUNK, CHUNK)]],
                         vrows.at[b], gsem.at[b])

    def wait_gather(b):
        pltpu.make_async_copy(
            wform_hbm.at[pl.ds(0, CHUNK)], frows.at[b], gsem.at[b]).wait()
        pltpu.make_async_copy(
            wvice_hbm.at[pl.ds(0, CHUNK)], vrows.at[b], gsem.at[b]).wait()

    def fire_write(v, b):
        row0 = (base + v) * CHUNK
        pltpu.async_copy(frows.at[b],
                         out_hbm.at[pl.ds(row0, CHUNK), pl.ds(0, H)],
                         wsem.at[b])
        pltpu.async_copy(vrows.at[b],
                         out_hbm.at[pl.ds(row0, CHUNK), pl.ds(H, H)],
                         wsem.at[b])

    def wait_write(b):
        row0 = base * CHUNK
        pltpu.make_async_copy(
            frows.at[b], out_hbm.at[pl.ds(row0, CHUNK), pl.ds(0, H)],
            wsem.at[b]).wait()
        pltpu.make_async_copy(
            vrows.at[b], out_hbm.at[pl.ds(row0, CHUNK), pl.ds(H, H)],
            wsem.at[b]).wait()

    fire_gather(0, 0)
    fire_gather(1, 1)

    def body(i, carry):
        for b in range(NBUF):
            v = NBUF * i + b
            nb = (b + 2) % NBUF
            wait_gather(b)
            fire_write(v, b)

            @pl.when(v >= 2)
            def _():
                wait_write(nb)

            fire_gather(v + 2, nb)
        return carry

    lax.fori_loop(0, CPW // NBUF, body, 0)

    for v, b in ((CPW - 2, 0), (CPW - 1, 1)):
        wait_gather(b)
        fire_write(v, b)

    for b in range(NBUF):
        wait_write(b)


def kernel(form_idx, vice_idx, W_form, W_vice):
    fi = form_idx.astype(jnp.int32).reshape(N)
    vi = vice_idx.astype(jnp.int32).reshape(N)
    out = _emb_gather(fi, vi, W_form, W_vice)
    return out.reshape(B, L, 2 * H)


# speedup vs baseline: 1.2721x; 1.2721x over previous
import functools

import jax
import jax.numpy as jnp
from jax import lax
from jax.experimental import pallas as pl
from jax.experimental.pallas import tpu as pltpu
from jax.experimental.pallas import tpu_sc as plsc

B = 4096
L = 50
H = 64
N = B * L
CHUNK = 128
NCHUNK = N // CHUNK
NC = 2
NS = 16
NW = NC * NS
CPW = NCHUNK // NW
IPW = N // NW
NBUF = 4


@functools.partial(
    pl.kernel,
    out_type=jax.ShapeDtypeStruct((N, 2 * H), jnp.float32),
    mesh=plsc.VectorSubcoreMesh(core_axis_name="c", subcore_axis_name="s"),
    compiler_params=pltpu.CompilerParams(use_tc_tiling_on_sc=False),
    scratch_types=[
        pltpu.VMEM((IPW,), jnp.int32),
        pltpu.VMEM((IPW,), jnp.int32),
        pltpu.VMEM((NBUF, CHUNK, H), jnp.float32),
        pltpu.VMEM((NBUF, CHUNK, H), jnp.float32),
        pltpu.SemaphoreType.DMA((NBUF,)),
        pltpu.SemaphoreType.DMA((NBUF,)),
    ],
)
def _emb_gather(form_idx_hbm, vice_idx_hbm, wform_hbm, wvice_hbm, out_hbm,
                fidx_v, vidx_v, frows, vrows, gsem, wsem):
    wid = lax.axis_index("s") * NC + lax.axis_index("c")
    base = wid * CPW
    pltpu.sync_copy(form_idx_hbm.at[pl.ds(wid * IPW, IPW)], fidx_v)
    pltpu.sync_copy(vice_idx_hbm.at[pl.ds(wid * IPW, IPW)], vidx_v)

    def fire_gather(v, b):
        pltpu.async_copy(wform_hbm.at[fidx_v.at[pl.ds(v * CHUNK, CHUNK)]],
                         frows.at[b], gsem.at[b])
        pltpu.async_copy(wvice_hbm.at[vidx_v.at[pl.ds(v * CHUNK, CHUNK)]],
                         vrows.at[b], gsem.at[b])

    def wait_gather(b):
        pltpu.make_async_copy(
            wform_hbm.at[pl.ds(0, CHUNK)], frows.at[b], gsem.at[b]).wait()
        pltpu.make_async_copy(
            wvice_hbm.at[pl.ds(0, CHUNK)], vrows.at[b], gsem.at[b]).wait()

    def fire_write(v, b):
        row0 = (base + v) * CHUNK
        pltpu.async_copy(frows.at[b],
                         out_hbm.at[pl.ds(row0, CHUNK), pl.ds(0, H)],
                         wsem.at[b])
        pltpu.async_copy(vrows.at[b],
                         out_hbm.at[pl.ds(row0, CHUNK), pl.ds(H, H)],
                         wsem.at[b])

    def wait_write(b):
        row0 = base * CHUNK
        pltpu.make_async_copy(
            frows.at[b], out_hbm.at[pl.ds(row0, CHUNK), pl.ds(0, H)],
            wsem.at[b]).wait()
        pltpu.make_async_copy(
            vrows.at[b], out_hbm.at[pl.ds(row0, CHUNK), pl.ds(H, H)],
            wsem.at[b]).wait()

    fire_gather(0, 0)
    fire_gather(1, 1)

    def body(i, carry):
        for b in range(NBUF):
            v = NBUF * i + b
            nb = (b + 2) % NBUF
            wait_gather(b)
            fire_write(v, b)

            @pl.when(v >= 2)
            def _():
                wait_write(nb)

            fire_gather(v + 2, nb)
        return carry

    lax.fori_loop(0, CPW // NBUF, body, 0)

    for v, b in ((CPW - 2, 0), (CPW - 1, 1)):
        wait_gather(b)
        fire_write(v, b)

    for b in range(NBUF):
        wait_write(b)


def kernel(form_idx, vice_idx, W_form, W_vice):
    fi = form_idx.astype(jnp.int32).T.reshape(N)
    vi = vice_idx.astype(jnp.int32).T.reshape(N)
    out = _emb_gather(fi, vi, W_form, W_vice)
    return out.reshape(L, B, 2 * H).transpose(1, 0, 2)
